# Initial kernel scaffold; baseline (speedup 1.0000x reference)
import jax
import jax.numpy as jnp
from jax.experimental import pallas as pl


def kernel(users, posts, post_adjs, up_masking, W_user, b_user, W_post, b_post, W_gat0, a1_0, a2_0, W_gat1, a1_1, a2_1, Wp0, bp0, Wp1, bp1, Wu0, bu0, Wu1, bu1):
    raise NotImplementedError("write your pallas kernel here")


# speedup vs baseline: 1.5535x; 1.5535x over previous
import functools

import jax
import jax.numpy as jnp
from jax.experimental import pallas as pl

N_POSTS = 5000
N_USERS = 1024
R = 256
RU = 256

_NEG = -1e9


def _f32dot(a, b):
    return jnp.dot(a, b, preferred_element_type=jnp.float32)


def _prep0_kernel(posts_ref, Wp_ref, bp_ref, Wg_ref, a1_ref, a2_ref,
                  hW_ref, s1_ref, s2t_ref):
    p0 = _f32dot(posts_ref[...], Wp_ref[...]) + bp_ref[...]
    hW = _f32dot(p0, Wg_ref[...])
    hW_ref[...] = hW
    s1_ref[...] = _f32dot(hW, a1_ref[...])
    s2t_ref[...] = jax.lax.dot_general(
        a2_ref[...], hW, (((1,), (1,)), ((), ())),
        preferred_element_type=jnp.float32)


def _prep1_kernel(h_ref, Wg_ref, a1_ref, a2_ref, hW_ref, s1_ref, s2t_ref):
    hW = _f32dot(h_ref[...], Wg_ref[...])
    hW_ref[...] = hW
    s1_ref[...] = _f32dot(hW, a1_ref[...])
    s2t_ref[...] = jax.lax.dot_general(
        a2_ref[...], hW, (((1,), (1,)), ((), ())),
        preferred_element_type=jnp.float32)


def _prep(h, Wg, a1, a2, enc=None):
    d_in = h.shape[1]
    if enc is None:
        kern = _prep1_kernel
        extra = ()
    else:
        kern = _prep0_kernel
        extra = (enc[0], enc[1].reshape(1, -1))
    return pl.pallas_call(
        kern,
        out_shape=(
            jax.ShapeDtypeStruct((N_POSTS, 64), jnp.float32),
            jax.ShapeDtypeStruct((N_POSTS, 1), jnp.float32),
            jax.ShapeDtypeStruct((1, N_POSTS), jnp.float32),
        ),
    )(h, *extra, Wg, a1.reshape(-1, 1), a2.reshape(1, -1))


def _gat_kernel(adj_ref, s1_ref, s2t_ref, hW_ref, out_ref):
    e = s1_ref[...] + s2t_ref[...]
    e = jnp.maximum(e, 0.2 * e)
    e = jnp.where(adj_ref[...] > 0, e, _NEG)
    m = jnp.max(e, axis=1, keepdims=True)
    ex = jnp.exp(e - m)
    l = jnp.sum(ex, axis=1, keepdims=True)
    h = _f32dot(ex, hW_ref[...]) / l
    out_ref[...] = jnp.where(h > 0, h, jnp.exp(h) - 1.0)


def _gat2_kernel(adj_ref, s1_ref, s2t_ref, hW_ref, Wp0_ref, bp0_ref,
                 Wp1_ref, bp1_ref, out_ref, lbl_ref):
    e = s1_ref[...] + s2t_ref[...]
    e = jnp.maximum(e, 0.2 * e)
    e = jnp.where(adj_ref[...] > 0, e, _NEG)
    m = jnp.max(e, axis=1, keepdims=True)
    ex = jnp.exp(e - m)
    l = jnp.sum(ex, axis=1, keepdims=True)
    h = _f32dot(ex, hW_ref[...]) / l
    h = jnp.where(h > 0, h, jnp.exp(h) - 1.0)
    out_ref[...] = h
    z = jnp.maximum(_f32dot(h, Wp0_ref[...]) + bp0_ref[...], 0.0)
    lbl_ref[...] = _f32dot(z, Wp1_ref[...]) + bp1_ref[...]


def _gat_layer(adj, hW, s1, s2t):
    grid = pl.cdiv(N_POSTS, R)
    return pl.pallas_call(
        _gat_kernel,
        grid=(grid,),
        in_specs=[
            pl.BlockSpec((R, N_POSTS), lambda i: (i, 0)),
            pl.BlockSpec((R, 1), lambda i: (i, 0)),
            pl.BlockSpec((1, N_POSTS), lambda i: (0, 0)),
            pl.BlockSpec((N_POSTS, 64), lambda i: (0, 0)),
        ],
        out_specs=pl.BlockSpec((R, 64), lambda i: (i, 0)),
        out_shape=jax.ShapeDtypeStruct((N_POSTS, 64), jnp.float32),
    )(adj, s1, s2t, hW)


def _gat_layer2(adj, hW, s1, s2t, Wp0, bp0, Wp1, bp1):
    grid = pl.cdiv(N_POSTS, R)
    return pl.pallas_call(
        _gat2_kernel,
        grid=(grid,),
        in_specs=[
            pl.BlockSpec((R, N_POSTS), lambda i: (i, 0)),
            pl.BlockSpec((R, 1), lambda i: (i, 0)),
            pl.BlockSpec((1, N_POSTS), lambda i: (0, 0)),
            pl.BlockSpec((N_POSTS, 64), lambda i: (0, 0)),
            pl.BlockSpec((64, 64), lambda i: (0, 0)),
            pl.BlockSpec((1, 64), lambda i: (0, 0)),
            pl.BlockSpec((64, 1), lambda i: (0, 0)),
            pl.BlockSpec((1, 1), lambda i: (0, 0)),
        ],
        out_specs=(
            pl.BlockSpec((R, 64), lambda i: (i, 0)),
            pl.BlockSpec((R, 1), lambda i: (i, 0)),
        ),
        out_shape=(
            jax.ShapeDtypeStruct((N_POSTS, 64), jnp.float32),
            jax.ShapeDtypeStruct((N_POSTS, 1), jnp.float32),
        ),
    )(adj, s1, s2t, hW, Wp0, bp0.reshape(1, -1), Wp1, bp1.reshape(1, -1))


def _user_kernel(users_ref, um_ref, h2_ref, Wu_ref, bu_ref, Wu0_ref,
                 bu0_ref, Wu1_ref, bu1_ref, out_ref):
    u = _f32dot(users_ref[...], Wu_ref[...]) + bu_ref[...]
    um = um_ref[...]
    denom = jnp.sum(um, axis=1, keepdims=True) + 1e-9
    agg = _f32dot(um, h2_ref[...]) / denom
    z = (_f32dot(u, Wu0_ref[0:64, :]) + _f32dot(agg, Wu0_ref[64:128, :])
         + bu0_ref[...])
    z = jnp.maximum(z, 0.0)
    out_ref[...] = _f32dot(z, Wu1_ref[...]) + bu1_ref[...]


def _user_branch(users, um, h2, W_user, b_user, Wu0, bu0, Wu1, bu1):
    grid = N_USERS // RU
    return pl.pallas_call(
        _user_kernel,
        grid=(grid,),
        in_specs=[
            pl.BlockSpec((RU, 128), lambda i: (i, 0)),
            pl.BlockSpec((RU, N_POSTS), lambda i: (i, 0)),
            pl.BlockSpec((N_POSTS, 64), lambda i: (0, 0)),
            pl.BlockSpec((128, 64), lambda i: (0, 0)),
            pl.BlockSpec((1, 64), lambda i: (0, 0)),
            pl.BlockSpec((128, 128), lambda i: (0, 0)),
            pl.BlockSpec((1, 128), lambda i: (0, 0)),
            pl.BlockSpec((128, 1), lambda i: (0, 0)),
            pl.BlockSpec((1, 1), lambda i: (0, 0)),
        ],
        out_specs=pl.BlockSpec((RU, 1), lambda i: (i, 0)),
        out_shape=jax.ShapeDtypeStruct((N_USERS, 1), jnp.float32),
    )(users, um, h2, W_user, b_user.reshape(1, -1), Wu0,
      bu0.reshape(1, -1), Wu1, bu1.reshape(1, -1))


def kernel(users, posts, post_adjs, up_masking, W_user, b_user, W_post,
           b_post, W_gat0, a1_0, a2_0, W_gat1, a1_1, a2_1,
           Wp0, bp0, Wp1, bp1, Wu0, bu0, Wu1, bu1):
    hW0, s1_0, s2t_0 = _prep(posts, W_gat0, a1_0, a2_0, enc=(W_post, b_post))
    h1 = _gat_layer(post_adjs, hW0, s1_0, s2t_0)
    hW1, s1_1, s2t_1 = _prep(h1, W_gat1, a1_1, a2_1)
    h2, post_label = _gat_layer2(post_adjs, hW1, s1_1, s2t_1,
                                 Wp0, bp0, Wp1, bp1)
    user_label = _user_branch(users, up_masking, h2, W_user, b_user,
                              Wu0, bu0, Wu1, bu1)
    return (user_label, post_label)


# speedup vs baseline: 1.5926x; 1.0252x over previous
import jax
import jax.numpy as jnp
from jax.experimental import pallas as pl

N_POSTS = 5000
N_USERS = 1024
R = 512
RU = 256


def _f32dot(a, b):
    return jnp.dot(a, b, preferred_element_type=jnp.float32)


def _prep0_kernel(posts_ref, Wp_ref, bp_ref, Wg_ref, a1_ref, a2_ref,
                  hW_ref, s1_ref, s2_ref):
    p0 = _f32dot(posts_ref[...], Wp_ref[...]) + bp_ref[...]
    hW = _f32dot(p0, Wg_ref[...])
    hW_ref[...] = hW
    s1_ref[...] = _f32dot(hW, a1_ref[...])
    s2_ref[...] = _f32dot(hW, a2_ref[...])


def _prep1_kernel(h_ref, Wg_ref, a1_ref, a2_ref, hW_ref, s1_ref, s2_ref):
    hW = _f32dot(h_ref[...], Wg_ref[...])
    hW_ref[...] = hW
    s1_ref[...] = _f32dot(hW, a1_ref[...])
    s2_ref[...] = _f32dot(hW, a2_ref[...])


def _prep(h, Wg, a1, a2, enc=None):
    if enc is None:
        kern = _prep1_kernel
        extra = ()
    else:
        kern = _prep0_kernel
        extra = (enc[0], enc[1].reshape(1, -1))
    return pl.pallas_call(
        kern,
        out_shape=(
            jax.ShapeDtypeStruct((N_POSTS, 64), jnp.float32),
            jax.ShapeDtypeStruct((N_POSTS, 1), jnp.float32),
            jax.ShapeDtypeStruct((N_POSTS, 1), jnp.float32),
        ),
    )(h, *extra, Wg, a1.reshape(-1, 1), a2.reshape(-1, 1))


def _gat_core(mask, s1, s2t, hW):
    e = s1 + s2t
    e = jnp.maximum(e, 0.2 * e)
    e = jnp.where(mask, e, -1e9)
    m = jnp.max(e, axis=1, keepdims=True)
    ex = jnp.exp(e - m)
    l = jnp.sum(ex, axis=1, keepdims=True)
    alpha = ex / l
    h = _f32dot(alpha, hW)
    return jnp.where(h > 0, h, jnp.exp(h) - 1.0)


def _gat1_kernel(adj_ref, s1_ref, s2t_ref, hW_ref, out_ref, m8_ref):
    adj = adj_ref[...]
    m8_ref[...] = adj.astype(jnp.int8)
    out_ref[...] = _gat_core(adj > 0, s1_ref[...], s2t_ref[...], hW_ref[...])


def _gat2_kernel(m8_ref, s1_ref, s2t_ref, hW_ref, Wp0_ref, bp0_ref,
                 Wp1_ref, bp1_ref, out_ref, lbl_ref):
    h = _gat_core(m8_ref[...].astype(jnp.int32) > 0, s1_ref[...],
                  s2t_ref[...], hW_ref[...])
    out_ref[...] = h
    z = jnp.maximum(_f32dot(h, Wp0_ref[...]) + bp0_ref[...], 0.0)
    lbl_ref[...] = _f32dot(z, Wp1_ref[...]) + bp1_ref[...]


def _common_specs():
    return [
        pl.BlockSpec((R, 1), lambda i: (i, 0)),
        pl.BlockSpec((1, N_POSTS), lambda i: (0, 0)),
        pl.BlockSpec((N_POSTS, 64), lambda i: (0, 0)),
    ]


def _gat_layer1(adj, hW, s1, s2t):
    grid = pl.cdiv(N_POSTS, R)
    return pl.pallas_call(
        _gat1_kernel,
        grid=(grid,),
        in_specs=[pl.BlockSpec((R, N_POSTS), lambda i: (i, 0))]
        + _common_specs(),
        out_specs=(
            pl.BlockSpec((R, 64), lambda i: (i, 0)),
            pl.BlockSpec((R, N_POSTS), lambda i: (i, 0)),
        ),
        out_shape=(
            jax.ShapeDtypeStruct((N_POSTS, 64), jnp.float32),
            jax.ShapeDtypeStruct((N_POSTS, N_POSTS), jnp.int8),
        ),
    )(adj, s1, s2t, hW)


def _gat_layer2(m8, hW, s1, s2t, Wp0, bp0, Wp1, bp1):
    grid = pl.cdiv(N_POSTS, R)
    return pl.pallas_call(
        _gat2_kernel,
        grid=(grid,),
        in_specs=[pl.BlockSpec((R, N_POSTS), lambda i: (i, 0))]
        + _common_specs()
        + [
            pl.BlockSpec((64, 64), lambda i: (0, 0)),
            pl.BlockSpec((1, 64), lambda i: (0, 0)),
            pl.BlockSpec((64, 1), lambda i: (0, 0)),
            pl.BlockSpec((1, 1), lambda i: (0, 0)),
        ],
        out_specs=(
            pl.BlockSpec((R, 64), lambda i: (i, 0)),
            pl.BlockSpec((R, 1), lambda i: (i, 0)),
        ),
        out_shape=(
            jax.ShapeDtypeStruct((N_POSTS, 64), jnp.float32),
            jax.ShapeDtypeStruct((N_POSTS, 1), jnp.float32),
        ),
    )(m8, s1, s2t, hW, Wp0, bp0.reshape(1, -1), Wp1, bp1.reshape(1, -1))


def _user_kernel(users_ref, um_ref, h2_ref, Wu_ref, bu_ref, Wu0_ref,
                 bu0_ref, Wu1_ref, bu1_ref, out_ref):
    u = _f32dot(users_ref[...], Wu_ref[...]) + bu_ref[...]
    um = um_ref[...]
    denom = jnp.sum(um, axis=1, keepdims=True) + 1e-9
    agg = _f32dot(um, h2_ref[...]) / denom
    z = (_f32dot(u, Wu0_ref[0:64, :]) + _f32dot(agg, Wu0_ref[64:128, :])
         + bu0_ref[...])
    z = jnp.maximum(z, 0.0)
    out_ref[...] = _f32dot(z, Wu1_ref[...]) + bu1_ref[...]


def _user_branch(users, um, h2, W_user, b_user, Wu0, bu0, Wu1, bu1):
    grid = N_USERS // RU
    return pl.pallas_call(
        _user_kernel,
        grid=(grid,),
        in_specs=[
            pl.BlockSpec((RU, 128), lambda i: (i, 0)),
            pl.BlockSpec((RU, N_POSTS), lambda i: (i, 0)),
            pl.BlockSpec((N_POSTS, 64), lambda i: (0, 0)),
            pl.BlockSpec((128, 64), lambda i: (0, 0)),
            pl.BlockSpec((1, 64), lambda i: (0, 0)),
            pl.BlockSpec((128, 128), lambda i: (0, 0)),
            pl.BlockSpec((1, 128), lambda i: (0, 0)),
            pl.BlockSpec((128, 1), lambda i: (0, 0)),
            pl.BlockSpec((1, 1), lambda i: (0, 0)),
        ],
        out_specs=pl.BlockSpec((RU, 1), lambda i: (i, 0)),
        out_shape=jax.ShapeDtypeStruct((N_USERS, 1), jnp.float32),
    )(users, um, h2, W_user, b_user.reshape(1, -1), Wu0,
      bu0.reshape(1, -1), Wu1, bu1.reshape(1, -1))


def kernel(users, posts, post_adjs, up_masking, W_user, b_user, W_post,
           b_post, W_gat0, a1_0, a2_0, W_gat1, a1_1, a2_1,
           Wp0, bp0, Wp1, bp1, Wu0, bu0, Wu1, bu1):
    hW0, s1_0, s2_0 = _prep(posts, W_gat0, a1_0, a2_0, enc=(W_post, b_post))
    h1, m8 = _gat_layer1(post_adjs, hW0, s1_0, s2_0.reshape(1, N_POSTS))
    hW1, s1_1, s2_1 = _prep(h1, W_gat1, a1_1, a2_1)
    h2, post_label = _gat_layer2(m8, hW1, s1_1, s2_1.reshape(1, N_POSTS),
                                 Wp0, bp0, Wp1, bp1)
    user_label = _user_branch(users, up_masking, h2, W_user, b_user,
                              Wu0, bu0, Wu1, bu1)
    return (user_label, post_label)
